# Initial kernel scaffold; baseline (speedup 1.0000x reference)
import jax
import jax.numpy as jnp
from jax.experimental import pallas as pl


def kernel(positions, masses):
    raise NotImplementedError("write your pallas kernel here")


# speedup vs baseline: 1.0063x; 1.0063x over previous
import functools

import jax
import jax.numpy as jnp
from jax import lax
from jax.experimental import pallas as pl
from jax.experimental.pallas import tpu as pltpu
from jax.experimental.pallas import tpu_sc as plsc

N_PART = 8388608
GRID_N = 256
NCELL = GRID_N * GRID_N * GRID_N

NUM_CORES = 2
NUM_SUBCORES = 16
NW = NUM_CORES * NUM_SUBCORES
STRIP = N_PART // NW
BLK = 4096
NBLK = STRIP // BLK
CHUNKS = BLK // 16


def _binner_body(xs, ys, zs, ms, lin_out, w_out, x_v, y_v, z_v, m_v, lin_v, w_v):
    wid = lax.axis_index("s") * NUM_CORES + lax.axis_index("c")
    base0 = wid * STRIP
    dx = jnp.float32(20.0) / jnp.float32(255.0)

    def blk_body(b, carry):
        base = base0 + b * BLK
        pltpu.sync_copy(xs.at[pl.ds(base, BLK)], x_v)
        pltpu.sync_copy(ys.at[pl.ds(base, BLK)], y_v)
        pltpu.sync_copy(zs.at[pl.ds(base, BLK)], z_v)
        pltpu.sync_copy(ms.at[pl.ds(base, BLK)], m_v)

        def chunk(j, c):
            o = j * 16
            x = x_v[pl.ds(o, 16)]
            y = y_v[pl.ds(o, 16)]
            z = z_v[pl.ds(o, 16)]
            m = m_v[pl.ds(o, 16)]
            ix = ((x + 10.0) / dx + 0.5).astype(jnp.int32)
            iy = ((y + 10.0) / dx + 0.5).astype(jnp.int32)
            iz = ((z + 10.0) / dx + 0.5).astype(jnp.int32)
            ok = (
                (ix >= 0) & (ix < GRID_N)
                & (iy >= 0) & (iy < GRID_N)
                & (iz >= 0) & (iz < GRID_N)
            )
            w = jnp.where(ok, m, jnp.float32(0.0))
            icx = jnp.clip(ix, 0, GRID_N - 1)
            icy = jnp.clip(iy, 0, GRID_N - 1)
            icz = jnp.clip(iz, 0, GRID_N - 1)
            lin = (icx * GRID_N + icy) * GRID_N + icz
            lin_v[pl.ds(o, 16)] = lin
            w_v[pl.ds(o, 16)] = w
            return c

        lax.fori_loop(0, CHUNKS, chunk, 0)
        pltpu.sync_copy(lin_v, lin_out.at[pl.ds(base, BLK)])
        pltpu.sync_copy(w_v, w_out.at[pl.ds(base, BLK)])
        return carry

    lax.fori_loop(0, NBLK, blk_body, 0)


_binner = functools.partial(
    pl.kernel,
    out_type=[
        jax.ShapeDtypeStruct((N_PART,), jnp.int32),
        jax.ShapeDtypeStruct((N_PART,), jnp.float32),
    ],
    scratch_types=[
        pltpu.VMEM((BLK,), jnp.float32),
        pltpu.VMEM((BLK,), jnp.float32),
        pltpu.VMEM((BLK,), jnp.float32),
        pltpu.VMEM((BLK,), jnp.float32),
        pltpu.VMEM((BLK,), jnp.int32),
        pltpu.VMEM((BLK,), jnp.float32),
    ],
    mesh=plsc.VectorSubcoreMesh(core_axis_name="c", subcore_axis_name="s"),
)(_binner_body)


@jax.jit
def kernel(positions, masses):
    xs = positions[:, 0]
    ys = positions[:, 1]
    zs = positions[:, 2]
    lin, w = _binner(xs, ys, zs, masses)
    data = jnp.zeros((NCELL,), dtype=jnp.float32).at[lin].add(w)
    return data.reshape((GRID_N, GRID_N, GRID_N))


# speedup vs baseline: 1.8472x; 1.8356x over previous
import functools

import jax
import jax.numpy as jnp
from jax import lax
from jax.experimental import pallas as pl
from jax.experimental.pallas import tpu as pltpu
from jax.experimental.pallas import tpu_sc as plsc

N_PART = 8388608
GRID_N = 256
NCELL = GRID_N * GRID_N * GRID_N

NUM_CORES = 2
NUM_SUBCORES = 16
NW = NUM_CORES * NUM_SUBCORES
STRIP = N_PART // NW
BLK = 4096
NBLK = STRIP // BLK
CHUNKS = BLK // 16


def _binner_body(xs, ys, zs, ms, lin_out, w_out, x_v, y_v, z_v, m_v, lin_v, w_v):
    wid = lax.axis_index("s") * NUM_CORES + lax.axis_index("c")
    base0 = wid * STRIP
    dx = jnp.float32(20.0) / jnp.float32(255.0)

    def blk_body(b, carry):
        base = base0 + b * BLK
        pltpu.sync_copy(xs.at[pl.ds(base, BLK)], x_v)
        pltpu.sync_copy(ys.at[pl.ds(base, BLK)], y_v)
        pltpu.sync_copy(zs.at[pl.ds(base, BLK)], z_v)
        pltpu.sync_copy(ms.at[pl.ds(base, BLK)], m_v)

        def chunk(j, c):
            o = j * 16
            x = x_v[pl.ds(o, 16)]
            y = y_v[pl.ds(o, 16)]
            z = z_v[pl.ds(o, 16)]
            m = m_v[pl.ds(o, 16)]
            ix = ((x + 10.0) / dx + 0.5).astype(jnp.int32)
            iy = ((y + 10.0) / dx + 0.5).astype(jnp.int32)
            iz = ((z + 10.0) / dx + 0.5).astype(jnp.int32)
            ok = (
                (ix >= 0) & (ix < GRID_N)
                & (iy >= 0) & (iy < GRID_N)
                & (iz >= 0) & (iz < GRID_N)
            )
            w = jnp.where(ok, m, jnp.float32(0.0))
            icx = jnp.clip(ix, 0, GRID_N - 1)
            icy = jnp.clip(iy, 0, GRID_N - 1)
            icz = jnp.clip(iz, 0, GRID_N - 1)
            lin = (icx * GRID_N + icy) * GRID_N + icz
            lin_v[pl.ds(o, 16)] = lin
            w_v[pl.ds(o, 16)] = w
            return c

        lax.fori_loop(0, CHUNKS, chunk, 0)
        pltpu.sync_copy(lin_v, lin_out.at[pl.ds(base, BLK)])
        pltpu.sync_copy(w_v, w_out.at[pl.ds(base, BLK)])
        return carry

    lax.fori_loop(0, NBLK, blk_body, 0)


NSLAB = 16
SLAB = NCELL // NSLAB
NPASS = NSLAB // NUM_CORES
TSTRIP = N_PART // NUM_SUBCORES
ABLK = 4096
NABLK = TSTRIP // ABLK
NGROUP = 32
GSZ = ABLK // NGROUP
TSHARE = SLAB // NUM_SUBCORES
ZBLK = 4096


def _accum_body(lin_hbm, w_hbm, grid_hbm, slab, lin_v, w_v, idx_st, val_st,
                zero_v, out_v, sem):
    cid = lax.axis_index("c")
    tid = lax.axis_index("s")
    tbase = tid * TSTRIP

    def zchunk(j, c):
        zero_v[pl.ds(j * 16, 16)] = jnp.zeros((16,), jnp.float32)
        return c
    lax.fori_loop(0, ZBLK // 16, zchunk, 0)

    def pass_body(p, carry):
        slab_id = jnp.where(cid == 0, p, NSLAB - 1 - p)
        slab_base = slab_id * SLAB

        def zb(i, c):
            pltpu.sync_copy(zero_v, slab.at[pl.ds(tid * TSHARE + i * ZBLK, ZBLK)])
            return c
        lax.fori_loop(0, TSHARE // ZBLK, zb, 0)
        plsc.subcore_barrier()

        def blk(b, c):
            base = tbase + b * ABLK
            pltpu.sync_copy(lin_hbm.at[pl.ds(base, ABLK)], lin_v)
            pltpu.sync_copy(w_hbm.at[pl.ds(base, ABLK)], w_v)
            copies = []
            for g in range(NGROUP):
                def chunk(j, c2, g=g):
                    o = g * GSZ + j * 16
                    u = lin_v[pl.ds(o, 16)] - slab_base
                    msk = (u >> 20) == 0
                    idx = u & jnp.int32(SLAB - 1)
                    val = jnp.where(msk, w_v[pl.ds(o, 16)], jnp.float32(0.0))
                    idx_st[g, pl.ds(j * 16, 16)] = idx
                    val_st[g, pl.ds(j * 16, 16)] = val
                    return c2
                lax.fori_loop(0, GSZ // 16, chunk, 0)
                copies.append(
                    pltpu.async_copy(val_st.at[g], slab.at[idx_st.at[g]], sem,
                                     add=True))
            for cp in copies:
                cp.wait()
            return c
        lax.fori_loop(0, NABLK, blk, 0)
        plsc.subcore_barrier()

        def ob(i, c):
            off = tid * TSHARE + i * (TSHARE // 4)
            pltpu.sync_copy(slab.at[pl.ds(off, TSHARE // 4)], out_v)
            pltpu.sync_copy(out_v, grid_hbm.at[pl.ds(slab_base + off, TSHARE // 4)])
            return c
        lax.fori_loop(0, 4, ob, 0)
        plsc.subcore_barrier()
        return carry

    lax.fori_loop(0, NPASS, pass_body, 0)


_accum = functools.partial(
    pl.kernel,
    out_type=jax.ShapeDtypeStruct((NCELL,), jnp.float32),
    scratch_types=[
        pltpu.VMEM_SHARED((SLAB,), jnp.float32),
        pltpu.VMEM((ABLK,), jnp.int32),
        pltpu.VMEM((ABLK,), jnp.float32),
        pltpu.VMEM((NGROUP, GSZ), jnp.int32),
        pltpu.VMEM((NGROUP, GSZ), jnp.float32),
        pltpu.VMEM((ZBLK,), jnp.float32),
        pltpu.VMEM((TSHARE // 4,), jnp.float32),
        pltpu.SemaphoreType.DMA,
    ],
    mesh=plsc.VectorSubcoreMesh(core_axis_name="c", subcore_axis_name="s"),
)(_accum_body)


_binner = functools.partial(
    pl.kernel,
    out_type=[
        jax.ShapeDtypeStruct((N_PART,), jnp.int32),
        jax.ShapeDtypeStruct((N_PART,), jnp.float32),
    ],
    scratch_types=[
        pltpu.VMEM((BLK,), jnp.float32),
        pltpu.VMEM((BLK,), jnp.float32),
        pltpu.VMEM((BLK,), jnp.float32),
        pltpu.VMEM((BLK,), jnp.float32),
        pltpu.VMEM((BLK,), jnp.int32),
        pltpu.VMEM((BLK,), jnp.float32),
    ],
    mesh=plsc.VectorSubcoreMesh(core_axis_name="c", subcore_axis_name="s"),
)(_binner_body)


@jax.jit
def kernel(positions, masses):
    xs = positions[:, 0]
    ys = positions[:, 1]
    zs = positions[:, 2]
    lin, w = _binner(xs, ys, zs, masses)
    data = _accum(lin, w)
    return data.reshape((GRID_N, GRID_N, GRID_N))


# speedup vs baseline: 1.8478x; 1.0003x over previous
import functools

import jax
import jax.numpy as jnp
from jax import lax
from jax.experimental import pallas as pl
from jax.experimental.pallas import tpu as pltpu
from jax.experimental.pallas import tpu_sc as plsc

N_PART = 8388608
GRID_N = 256
NCELL = GRID_N * GRID_N * GRID_N

NUM_CORES = 2
NUM_SUBCORES = 16
NW = NUM_CORES * NUM_SUBCORES
STRIP = N_PART // NW
BLK = 4096
NBLK = STRIP // BLK
CHUNKS = BLK // 16


def _binner_body(xs, ys, zs, ms, lin_out, w_out, x_v, y_v, z_v, m_v, lin_v, w_v):
    wid = lax.axis_index("s") * NUM_CORES + lax.axis_index("c")
    base0 = wid * STRIP
    dx = jnp.float32(20.0) / jnp.float32(255.0)

    def blk_body(b, carry):
        base = base0 + b * BLK
        pltpu.sync_copy(xs.at[pl.ds(base, BLK)], x_v)
        pltpu.sync_copy(ys.at[pl.ds(base, BLK)], y_v)
        pltpu.sync_copy(zs.at[pl.ds(base, BLK)], z_v)
        pltpu.sync_copy(ms.at[pl.ds(base, BLK)], m_v)

        def chunk(j, c):
            o = j * 16
            x = x_v[pl.ds(o, 16)]
            y = y_v[pl.ds(o, 16)]
            z = z_v[pl.ds(o, 16)]
            m = m_v[pl.ds(o, 16)]
            ix = ((x + 10.0) / dx + 0.5).astype(jnp.int32)
            iy = ((y + 10.0) / dx + 0.5).astype(jnp.int32)
            iz = ((z + 10.0) / dx + 0.5).astype(jnp.int32)
            ok = (
                (ix >= 0) & (ix < GRID_N)
                & (iy >= 0) & (iy < GRID_N)
                & (iz >= 0) & (iz < GRID_N)
            )
            w = jnp.where(ok, m, jnp.float32(0.0))
            icx = jnp.clip(ix, 0, GRID_N - 1)
            icy = jnp.clip(iy, 0, GRID_N - 1)
            icz = jnp.clip(iz, 0, GRID_N - 1)
            lin = (icx * GRID_N + icy) * GRID_N + icz
            lin_v[pl.ds(o, 16)] = lin
            w_v[pl.ds(o, 16)] = w
            return c

        lax.fori_loop(0, CHUNKS, chunk, 0)
        pltpu.sync_copy(lin_v, lin_out.at[pl.ds(base, BLK)])
        pltpu.sync_copy(w_v, w_out.at[pl.ds(base, BLK)])
        return carry

    lax.fori_loop(0, NBLK, blk_body, 0)


NSLAB = 16
SLAB = NCELL // NSLAB
NPASS = NSLAB // NUM_CORES
TSTRIP = N_PART // NUM_SUBCORES
ABLK = 4096
NABLK = TSTRIP // ABLK
NGROUP = 32
GSZ = ABLK // NGROUP
TSHARE = SLAB // NUM_SUBCORES
ZBLK = 4096


def _accum_body(lin_hbm, w_hbm, grid_hbm, slab, lin_v, w_v, idx_st, val_st,
                zero_v, out_v, sem):
    cid = lax.axis_index("c")
    tid = lax.axis_index("s")
    tbase = tid * TSTRIP

    def zchunk(j, c):
        zero_v[pl.ds(j * 16, 16)] = jnp.zeros((16,), jnp.float32)
        return c
    lax.fori_loop(0, ZBLK // 16, zchunk, 0)

    def pass_body(p, carry):
        slab_id = jnp.where(cid == 0, p, NSLAB - 1 - p)
        slab_base = slab_id * SLAB

        def zb(i, c):
            pltpu.sync_copy(zero_v, slab.at[pl.ds(tid * TSHARE + i * ZBLK, ZBLK)])
            return c
        lax.fori_loop(0, TSHARE // ZBLK, zb, 0)
        plsc.subcore_barrier()

        def blk(b, c):
            base = tbase + b * ABLK
            pltpu.sync_copy(lin_hbm.at[pl.ds(base, ABLK)], lin_v)
            pltpu.sync_copy(w_hbm.at[pl.ds(base, ABLK)], w_v)
            copies = []
            for g in range(NGROUP):
                for j in range(GSZ // 16):
                    o = g * GSZ + j * 16
                    u = lin_v[pl.ds(o, 16)] - slab_base
                    idx = u & jnp.int32(SLAB - 1)
                    val = jnp.where(idx == u, w_v[pl.ds(o, 16)],
                                    jnp.float32(0.0))
                    idx_st[g, pl.ds(j * 16, 16)] = idx
                    val_st[g, pl.ds(j * 16, 16)] = val
                copies.append(
                    pltpu.async_copy(val_st.at[g], slab.at[idx_st.at[g]], sem,
                                     add=True))
            for cp in copies:
                cp.wait()
            return c
        lax.fori_loop(0, NABLK, blk, 0)
        plsc.subcore_barrier()

        def ob(i, c):
            off = tid * TSHARE + i * (TSHARE // 4)
            pltpu.sync_copy(slab.at[pl.ds(off, TSHARE // 4)], out_v)
            pltpu.sync_copy(out_v, grid_hbm.at[pl.ds(slab_base + off, TSHARE // 4)])
            return c
        lax.fori_loop(0, 4, ob, 0)
        plsc.subcore_barrier()
        return carry

    lax.fori_loop(0, NPASS, pass_body, 0)


_accum = functools.partial(
    pl.kernel,
    out_type=jax.ShapeDtypeStruct((NCELL,), jnp.float32),
    scratch_types=[
        pltpu.VMEM_SHARED((SLAB,), jnp.float32),
        pltpu.VMEM((ABLK,), jnp.int32),
        pltpu.VMEM((ABLK,), jnp.float32),
        pltpu.VMEM((NGROUP, GSZ), jnp.int32),
        pltpu.VMEM((NGROUP, GSZ), jnp.float32),
        pltpu.VMEM((ZBLK,), jnp.float32),
        pltpu.VMEM((TSHARE // 4,), jnp.float32),
        pltpu.SemaphoreType.DMA,
    ],
    mesh=plsc.VectorSubcoreMesh(core_axis_name="c", subcore_axis_name="s"),
)(_accum_body)


_binner = functools.partial(
    pl.kernel,
    out_type=[
        jax.ShapeDtypeStruct((N_PART,), jnp.int32),
        jax.ShapeDtypeStruct((N_PART,), jnp.float32),
    ],
    scratch_types=[
        pltpu.VMEM((BLK,), jnp.float32),
        pltpu.VMEM((BLK,), jnp.float32),
        pltpu.VMEM((BLK,), jnp.float32),
        pltpu.VMEM((BLK,), jnp.float32),
        pltpu.VMEM((BLK,), jnp.int32),
        pltpu.VMEM((BLK,), jnp.float32),
    ],
    mesh=plsc.VectorSubcoreMesh(core_axis_name="c", subcore_axis_name="s"),
)(_binner_body)


@jax.jit
def kernel(positions, masses):
    xs = positions[:, 0]
    ys = positions[:, 1]
    zs = positions[:, 2]
    lin, w = _binner(xs, ys, zs, masses)
    data = _accum(lin, w)
    return data.reshape((GRID_N, GRID_N, GRID_N))
